# Initial kernel scaffold; baseline (speedup 1.0000x reference)
import jax
import jax.numpy as jnp
from jax.experimental import pallas as pl


def kernel(bu_v, w_bu, t, i_act_nb):
    raise NotImplementedError("write your pallas kernel here")


# speedup vs baseline: 1.6371x; 1.6371x over previous
import functools

import jax
import jax.numpy as jnp
from jax.experimental import pallas as pl
from jax.experimental.pallas import tpu as pltpu

_H, _W = 64, 128
_D = 256
_SIGMA0 = 8.0
_LR0 = 0.1
_TAU = 1000.0
_BIG_I32 = 2**30


def _som_body(x_ref, w_ref, t_ref, iact_ref,
              w_out_ref, iact_out_ref, idx_ref, qe_ref, te_ref):
    w = w_ref[...]
    x = x_ref[...]
    diff = x - w
    d2 = jnp.sum(diff * diff, axis=2)

    ii = jax.lax.broadcasted_iota(jnp.int32, (_H, _W), 0)
    jj = jax.lax.broadcasted_iota(jnp.int32, (_H, _W), 1)
    lin = ii * _W + jj

    minval = jnp.min(d2, keepdims=True)
    idx = jnp.min(jnp.where(d2 == minval, lin, _BIG_I32), keepdims=True)

    maxval = jnp.max(d2, keepdims=True)
    d2b = jnp.where(lin == idx, maxval, d2)
    minval2 = jnp.min(d2b, keepdims=True)
    idx2 = jnp.min(jnp.where(d2b == minval2, lin, _BIG_I32), keepdims=True)

    xw = idx // _W
    yw = idx % _W

    dif = (ii - xw).astype(jnp.float32)
    djf = (jj - yw).astype(jnp.float32)
    d2map = dif * dif + djf * djf

    tf = t_ref[0].astype(jnp.float32)
    decay = jnp.exp(jnp.full((1, 1), -tf / _TAU, jnp.float32))
    sigma2x2 = 2.0 * _SIGMA0 * _SIGMA0 * decay * decay
    alpha = (_LR0 * decay) * jnp.exp(-d2map / sigma2x2)

    w_out_ref[...] = w + alpha[:, :, None] * diff
    iact_out_ref[...] = iact_ref[...] + (lin == idx).astype(jnp.int32)

    idx_ref[...] = jnp.concatenate([idx, idx2], axis=1)
    qe_ref[...] = minval
    te_ref[...] = jnp.sqrt(jnp.max(jnp.where(lin == idx2, d2map, -1.0),
                                   keepdims=True))


@functools.partial(jax.jit, static_argnames=())
def kernel(bu_v, w_bu, t, i_act_nb):
    x = bu_v.reshape(1, 1, _D)
    t_s = t.reshape(1)
    outs = pl.pallas_call(
        _som_body,
        out_shape=(
            jax.ShapeDtypeStruct((_H, _W, _D), jnp.float32),
            jax.ShapeDtypeStruct((_H, _W), jnp.int32),
            jax.ShapeDtypeStruct((1, 2), jnp.int32),
            jax.ShapeDtypeStruct((1, 1), jnp.float32),
            jax.ShapeDtypeStruct((1, 1), jnp.float32),
        ),
        in_specs=[
            pl.BlockSpec(memory_space=pltpu.VMEM),
            pl.BlockSpec(memory_space=pltpu.VMEM),
            pl.BlockSpec(memory_space=pltpu.SMEM),
            pl.BlockSpec(memory_space=pltpu.VMEM),
        ],
        out_specs=(
            pl.BlockSpec(memory_space=pltpu.VMEM),
            pl.BlockSpec(memory_space=pltpu.VMEM),
            pl.BlockSpec(memory_space=pltpu.VMEM),
            pl.BlockSpec(memory_space=pltpu.VMEM),
            pl.BlockSpec(memory_space=pltpu.VMEM),
        ),
    )(x, w_bu, t_s, i_act_nb)
    new_w, new_iact, idxs, qe, te = outs
    idx = idxs[0, 0]
    winner = jnp.stack([idx // _W, idx % _W]).astype(jnp.int32)
    return (new_w, winner, new_iact, t + 1,
            qe.reshape(()), te.reshape(()))


# speedup vs baseline: 1.9290x; 1.1783x over previous
import jax
import jax.numpy as jnp
from jax.experimental import pallas as pl
from jax.experimental.pallas import tpu as pltpu

_H, _W = 64, 128
_D = 256
_NB = 8
_BH = _H // _NB
_SIGMA0 = 8.0
_LR0 = 0.1
_TAU = 1000.0
_BIG_I32 = 2**30


def _som_body(x_ref, w_ref, t_ref, iact_ref,
              w_out_ref, iact_out_ref, winner_ref, qe_ref, te_ref, t1_ref,
              w_save, d2_save, scr_i, scr_f):
    s = pl.program_id(0)

    @pl.when(s < _NB)
    def _phase_dist():
        w = w_ref[...]
        diff = x_ref[...] - w
        d2 = jnp.sum(diff * diff, axis=2)
        w_save[pl.ds(s * _BH, _BH), :, :] = w
        d2_save[pl.ds(s * _BH, _BH), :] = d2

    @pl.when(s == _NB)
    def _reduce():
        d2 = d2_save[...]
        ii = jax.lax.broadcasted_iota(jnp.int32, (_H, _W), 0)
        jj = jax.lax.broadcasted_iota(jnp.int32, (_H, _W), 1)
        lin = ii * _W + jj

        minval = jnp.min(d2, keepdims=True)
        idx = jnp.min(jnp.where(d2 == minval, lin, _BIG_I32), keepdims=True)
        maxval = jnp.max(d2, keepdims=True)
        d2b = jnp.where(lin == idx, maxval, d2)
        minval2 = jnp.min(d2b, keepdims=True)
        idx2 = jnp.min(jnp.where(d2b == minval2, lin, _BIG_I32), keepdims=True)

        xw = idx // _W
        yw = idx % _W
        dif = (ii - xw).astype(jnp.float32)
        djf = (jj - yw).astype(jnp.float32)
        d2map = dif * dif + djf * djf

        tf = t_ref[0].astype(jnp.float32)
        lr = _LR0 * jnp.exp(jnp.full((1, 1), -tf / _TAU, jnp.float32))
        inv2s2 = (jnp.exp(jnp.full((1, 1), 2.0 * tf / _TAU, jnp.float32))
                  * (1.0 / (2.0 * _SIGMA0 * _SIGMA0)))

        scr_i[...] = jnp.concatenate([xw, yw], axis=1)
        scr_f[...] = jnp.concatenate([lr, inv2s2], axis=1)

        winner_ref[...] = jnp.concatenate([xw, yw], axis=1)
        qe_ref[...] = minval
        te_ref[...] = jnp.sqrt(jnp.max(jnp.where(lin == idx2, d2map, -1.0),
                                       keepdims=True))
        iact_out_ref[...] = iact_ref[...] + (lin == idx).astype(jnp.int32)
        t1_ref[0] = t_ref[0] + 1

    @pl.when(s >= _NB)
    def _phase_update():
        b = s - _NB
        w = w_save[pl.ds(b * _BH, _BH), :, :]
        xw = scr_i[0:1, 0:1]
        yw = scr_i[0:1, 1:2]
        lr = scr_f[0:1, 0:1]
        inv2s2 = scr_f[0:1, 1:2]
        ii = jax.lax.broadcasted_iota(jnp.int32, (_BH, _W), 0) + b * _BH
        jj = jax.lax.broadcasted_iota(jnp.int32, (_BH, _W), 1)
        dif = (ii - xw).astype(jnp.float32)
        djf = (jj - yw).astype(jnp.float32)
        d2map = dif * dif + djf * djf
        alpha = lr * jnp.exp(-d2map * inv2s2)
        w_out_ref[...] = w + alpha[:, :, None] * (x_ref[...] - w)


def kernel(bu_v, w_bu, t, i_act_nb):
    x = bu_v.reshape(1, 1, _D)
    t_s = t.reshape(1)
    outs = pl.pallas_call(
        _som_body,
        grid=(2 * _NB,),
        out_shape=(
            jax.ShapeDtypeStruct((_H, _W, _D), jnp.float32),
            jax.ShapeDtypeStruct((_H, _W), jnp.int32),
            jax.ShapeDtypeStruct((1, 2), jnp.int32),
            jax.ShapeDtypeStruct((1, 1), jnp.float32),
            jax.ShapeDtypeStruct((1, 1), jnp.float32),
            jax.ShapeDtypeStruct((1,), jnp.int32),
        ),
        in_specs=[
            pl.BlockSpec((1, 1, _D), lambda s: (0, 0, 0)),
            pl.BlockSpec((_BH, _W, _D),
                         lambda s: (jnp.minimum(s, _NB - 1), 0, 0)),
            pl.BlockSpec(memory_space=pltpu.SMEM),
            pl.BlockSpec((_H, _W), lambda s: (0, 0)),
        ],
        out_specs=(
            pl.BlockSpec((_BH, _W, _D),
                         lambda s: (jnp.maximum(s - _NB, 0), 0, 0)),
            pl.BlockSpec((_H, _W), lambda s: (0, 0)),
            pl.BlockSpec((1, 2), lambda s: (0, 0)),
            pl.BlockSpec((1, 1), lambda s: (0, 0)),
            pl.BlockSpec((1, 1), lambda s: (0, 0)),
            pl.BlockSpec(memory_space=pltpu.SMEM),
        ),
        scratch_shapes=[
            pltpu.VMEM((_H, _W, _D), jnp.float32),
            pltpu.VMEM((_H, _W), jnp.float32),
            pltpu.VMEM((1, 2), jnp.int32),
            pltpu.VMEM((1, 2), jnp.float32),
        ],
    )(x, w_bu, t_s, i_act_nb)
    new_w, new_iact, winner2, qe, te, t1 = outs
    return (new_w, winner2.reshape(2), new_iact, t1.reshape(()),
            qe.reshape(()), te.reshape(()))


# speedup vs baseline: 2.6927x; 1.3959x over previous
import jax
import jax.numpy as jnp
from jax.experimental import pallas as pl
from jax.experimental.pallas import tpu as pltpu

_H, _W = 64, 128
_D = 256
_NB = 8
_BH = _H // _NB
_SIGMA0 = 8.0
_LR0 = 0.1
_TAU = 1000.0
_BIG_I32 = 2**30


def _som_body(x_ref, w_hbm, t_ref, iact_ref,
              w_out_hbm, iact_out_ref, winner_ref, qe_ref, te_ref, t1_ref,
              w_vmem, sem_in, sem_out):
    copies_in = [
        pltpu.make_async_copy(w_hbm.at[pl.ds(b * _BH, _BH)],
                              w_vmem.at[pl.ds(b * _BH, _BH)],
                              sem_in.at[b])
        for b in range(_NB)
    ]
    for c in copies_in:
        c.start()

    x = x_ref[...]

    d2_parts = []
    for b in range(_NB):
        copies_in[b].wait()
        w = w_vmem[pl.ds(b * _BH, _BH), :, :]
        diff = x - w
        d2_parts.append(jnp.sum(diff * diff, axis=2))
    d2 = jnp.concatenate(d2_parts, axis=0)

    ii = jax.lax.broadcasted_iota(jnp.int32, (_H, _W), 0)
    jj = jax.lax.broadcasted_iota(jnp.int32, (_H, _W), 1)
    lin = ii * _W + jj

    minval = jnp.min(d2, keepdims=True)
    idx = jnp.min(jnp.where(d2 == minval, lin, _BIG_I32), keepdims=True)
    maxval = jnp.max(d2, keepdims=True)
    d2b = jnp.where(lin == idx, maxval, d2)
    minval2 = jnp.min(d2b, keepdims=True)
    idx2 = jnp.min(jnp.where(d2b == minval2, lin, _BIG_I32), keepdims=True)

    xw = idx // _W
    yw = idx % _W
    dif = (ii - xw).astype(jnp.float32)
    djf = (jj - yw).astype(jnp.float32)
    d2map = dif * dif + djf * djf

    tf = t_ref[0].astype(jnp.float32)
    lr = _LR0 * jnp.exp(jnp.full((1, 1), -tf / _TAU, jnp.float32))
    inv2s2 = (jnp.exp(jnp.full((1, 1), 2.0 * tf / _TAU, jnp.float32))
              * (1.0 / (2.0 * _SIGMA0 * _SIGMA0)))
    alpha = lr * jnp.exp(-d2map * inv2s2)

    winner_ref[...] = jnp.concatenate([xw, yw], axis=1)
    qe_ref[...] = minval
    te_ref[...] = jnp.sqrt(jnp.max(jnp.where(lin == idx2, d2map, -1.0),
                                   keepdims=True))
    iact_out_ref[...] = iact_ref[...] + (lin == idx).astype(jnp.int32)
    t1_ref[0] = t_ref[0] + 1

    copies_out = []
    for b in range(_NB):
        w = w_vmem[pl.ds(b * _BH, _BH), :, :]
        a = alpha[b * _BH:(b + 1) * _BH, :]
        w_vmem[pl.ds(b * _BH, _BH), :, :] = w + a[:, :, None] * (x - w)
        c = pltpu.make_async_copy(w_vmem.at[pl.ds(b * _BH, _BH)],
                                  w_out_hbm.at[pl.ds(b * _BH, _BH)],
                                  sem_out.at[b])
        c.start()
        copies_out.append(c)
    for c in copies_out:
        c.wait()


def kernel(bu_v, w_bu, t, i_act_nb):
    x = bu_v.reshape(1, 1, _D)
    t_s = t.reshape(1)
    outs = pl.pallas_call(
        _som_body,
        out_shape=(
            jax.ShapeDtypeStruct((_H, _W, _D), jnp.float32),
            jax.ShapeDtypeStruct((_H, _W), jnp.int32),
            jax.ShapeDtypeStruct((1, 2), jnp.int32),
            jax.ShapeDtypeStruct((1, 1), jnp.float32),
            jax.ShapeDtypeStruct((1, 1), jnp.float32),
            jax.ShapeDtypeStruct((1,), jnp.int32),
        ),
        in_specs=[
            pl.BlockSpec(memory_space=pltpu.VMEM),
            pl.BlockSpec(memory_space=pl.MemorySpace.ANY),
            pl.BlockSpec(memory_space=pltpu.SMEM),
            pl.BlockSpec(memory_space=pltpu.VMEM),
        ],
        out_specs=(
            pl.BlockSpec(memory_space=pl.MemorySpace.ANY),
            pl.BlockSpec(memory_space=pltpu.VMEM),
            pl.BlockSpec(memory_space=pltpu.VMEM),
            pl.BlockSpec(memory_space=pltpu.VMEM),
            pl.BlockSpec(memory_space=pltpu.VMEM),
            pl.BlockSpec(memory_space=pltpu.SMEM),
        ),
        scratch_shapes=[
            pltpu.VMEM((_H, _W, _D), jnp.float32),
            pltpu.SemaphoreType.DMA((_NB,)),
            pltpu.SemaphoreType.DMA((_NB,)),
        ],
    )(x, w_bu, t_s, i_act_nb)
    new_w, new_iact, winner2, qe, te, t1 = outs
    return (new_w, winner2.reshape(2), new_iact, t1.reshape(()),
            qe.reshape(()), te.reshape(()))
